# baseline (speedup 1.0000x reference)
import jax
import jax.numpy as jnp
from jax.experimental import pallas as pl

BN = 128


def _vq_kernel(t_ref, w_ref, tsq_ref, csq_ref, out_ref):
    k = w_ref.shape[0]
    t = t_ref[...]
    g = jax.lax.dot_general(
        t, w_ref[...], (((1,), (1,)), ((), ())),
        preferred_element_type=jnp.float32)
    d2 = (tsq_ref[...] + csq_ref[...]) + g
    dist = jnp.sqrt(jnp.maximum(d2, 0.0))
    r = jnp.min(dist, axis=1, keepdims=True)
    iota = jax.lax.broadcasted_iota(jnp.int32, dist.shape, 1)
    label = jnp.min(jnp.where(dist <= r, iota, k),
                    axis=1, keepdims=True)
    out_ref[...] = jnp.where(iota == label, 1.0, 0.0).astype(jnp.float32)


def kernel(target, codebook, counts):
    n, d = target.shape
    k = codebook.shape[0]
    cb = codebook / counts[:, None]
    t_sq = jnp.sum(target * target, axis=1, keepdims=True)
    c_sq = jnp.sum(cb * cb, axis=1)[None, :]
    w = -2.0 * cb
    return pl.pallas_call(
        _vq_kernel,
        grid=(n // BN,),
        in_specs=[
            pl.BlockSpec((BN, d), lambda i: (i, 0)),
            pl.BlockSpec((k, d), lambda i: (0, 0)),
            pl.BlockSpec((BN, 1), lambda i: (i, 0)),
            pl.BlockSpec((1, k), lambda i: (0, 0)),
        ],
        out_specs=pl.BlockSpec((BN, k), lambda i: (i, 0)),
        out_shape=jax.ShapeDtypeStruct((n, k), jnp.float32),
    )(target, w, t_sq, c_sq)


# speedup vs baseline: 1.0411x; 1.0411x over previous
import jax
import jax.numpy as jnp
from jax.experimental import pallas as pl

BN = 128


def _sqrt_ulp_threshold(r):
    bits = jax.lax.bitcast_convert_type(r, jnp.int32)
    rp = jax.lax.bitcast_convert_type(bits + 1, jnp.float32)
    u = rp - r
    h = 0.5 * u
    c = 4097.0 * r
    r_hi = c - (c - r)
    r_lo = r - r_hi
    p_hi = r * r
    p_lo = ((r_hi * r_hi - p_hi) + 2.0 * (r_hi * r_lo)) + r_lo * r_lo
    rh = r * u
    h2 = h * h
    s1 = p_lo + rh
    z = p_hi + (s1 + h2)
    resid = ((p_hi - z) + s1) + h2
    zbits = jax.lax.bitcast_convert_type(z, jnp.int32)
    z_prev = jax.lax.bitcast_convert_type(zbits - 1, jnp.float32)
    t = jnp.where(resid > 0, z, z_prev)
    return jnp.where(r == 0, jnp.float32(0.0), t)


def _device_sqrt_threshold(r):
    tc = _sqrt_ulp_threshold(r)
    tbits = jax.lax.bitcast_convert_type(tc, jnp.int32)
    t = jnp.full_like(r, -jnp.inf)
    for i in range(-8, 9):
        c = jax.lax.bitcast_convert_type(tbits + i, jnp.float32)
        ok = jnp.sqrt(jnp.maximum(c, 0.0)) <= r
        t = jnp.where(ok, c, t)
    return jnp.where(r == 0, jnp.float32(0.0), t)


def _vq_kernel(t_ref, w_ref, tsq_ref, csq_ref, out_ref):
    k = w_ref.shape[0]
    t = t_ref[...]
    g = jax.lax.dot_general(
        t, w_ref[...], (((1,), (1,)), ((), ())),
        preferred_element_type=jnp.float32)
    d2 = (tsq_ref[...] + csq_ref[...]) + g
    m = jnp.min(d2, axis=1, keepdims=True)
    r = jnp.sqrt(jnp.maximum(m, 0.0))
    thresh = _device_sqrt_threshold(r)
    iota = jax.lax.broadcasted_iota(jnp.int32, d2.shape, 1)
    label = jnp.min(jnp.where(d2 <= thresh, iota, k),
                    axis=1, keepdims=True)
    out_ref[...] = jnp.where(iota == label, 1.0, 0.0).astype(jnp.float32)


def kernel(target, codebook, counts):
    n, d = target.shape
    k = codebook.shape[0]
    cb = codebook / counts[:, None]
    t_sq = jnp.sum(target * target, axis=1, keepdims=True)
    c_sq = jnp.sum(cb * cb, axis=1)[None, :]
    w = -2.0 * cb
    return pl.pallas_call(
        _vq_kernel,
        grid=(n // BN,),
        in_specs=[
            pl.BlockSpec((BN, d), lambda i: (i, 0)),
            pl.BlockSpec((k, d), lambda i: (0, 0)),
            pl.BlockSpec((BN, 1), lambda i: (i, 0)),
            pl.BlockSpec((1, k), lambda i: (0, 0)),
        ],
        out_specs=pl.BlockSpec((BN, k), lambda i: (i, 0)),
        out_shape=jax.ShapeDtypeStruct((n, k), jnp.float32),
    )(target, w, t_sq, c_sq)


# speedup vs baseline: 1.1033x; 1.0598x over previous
import jax
import jax.numpy as jnp
from jax.experimental import pallas as pl

BN = 128


def _sqrt_ulp_threshold(r):
    bits = jax.lax.bitcast_convert_type(r, jnp.int32)
    rp = jax.lax.bitcast_convert_type(bits + 1, jnp.float32)
    u = rp - r
    h = 0.5 * u
    c = 4097.0 * r
    r_hi = c - (c - r)
    r_lo = r - r_hi
    p_hi = r * r
    p_lo = ((r_hi * r_hi - p_hi) + 2.0 * (r_hi * r_lo)) + r_lo * r_lo
    rh = r * u
    h2 = h * h
    s1 = p_lo + rh
    z = p_hi + (s1 + h2)
    resid = ((p_hi - z) + s1) + h2
    zbits = jax.lax.bitcast_convert_type(z, jnp.int32)
    z_prev = jax.lax.bitcast_convert_type(zbits - 1, jnp.float32)
    t = jnp.where(resid > 0, z, z_prev)
    return jnp.where(r == 0, jnp.float32(0.0), t)


def _device_sqrt_threshold(r):
    bn = r.shape[0]
    tc = _sqrt_ulp_threshold(r)
    tbits = jax.lax.bitcast_convert_type(tc, jnp.int32)
    off = jax.lax.broadcasted_iota(jnp.int32, (bn, 128), 1) - 63
    c = jax.lax.bitcast_convert_type(tbits + off, jnp.float32)
    ok = jnp.sqrt(jnp.maximum(c, 0.0)) <= r
    t = jnp.max(jnp.where(ok, c, -jnp.inf), axis=1, keepdims=True)
    return jnp.where(r == 0, jnp.float32(0.0), t)


def _vq_kernel(t_ref, w_ref, tsq_ref, csq_ref, out_ref):
    k = w_ref.shape[0]
    t = t_ref[...]
    g = jax.lax.dot_general(
        t, w_ref[...], (((1,), (1,)), ((), ())),
        preferred_element_type=jnp.float32)
    d2 = (tsq_ref[...] + csq_ref[...]) + g
    m = jnp.min(d2, axis=1, keepdims=True)
    r = jnp.sqrt(jnp.maximum(m, 0.0))
    thresh = _device_sqrt_threshold(r)
    iota = jax.lax.broadcasted_iota(jnp.int32, d2.shape, 1)
    label = jnp.min(jnp.where(d2 <= thresh, iota, k),
                    axis=1, keepdims=True)
    out_ref[...] = jnp.where(iota == label, 1.0, 0.0).astype(jnp.float32)


def kernel(target, codebook, counts):
    n, d = target.shape
    k = codebook.shape[0]
    cb = codebook / counts[:, None]
    t_sq = jnp.sum(target * target, axis=1, keepdims=True)
    c_sq = jnp.sum(cb * cb, axis=1)[None, :]
    w = -2.0 * cb
    return pl.pallas_call(
        _vq_kernel,
        grid=(n // BN,),
        in_specs=[
            pl.BlockSpec((BN, d), lambda i: (i, 0)),
            pl.BlockSpec((k, d), lambda i: (0, 0)),
            pl.BlockSpec((BN, 1), lambda i: (i, 0)),
            pl.BlockSpec((1, k), lambda i: (0, 0)),
        ],
        out_specs=pl.BlockSpec((BN, k), lambda i: (i, 0)),
        out_shape=jax.ShapeDtypeStruct((n, k), jnp.float32),
    )(target, w, t_sq, c_sq)


# speedup vs baseline: 1.5316x; 1.3882x over previous
import jax
import jax.numpy as jnp
from jax.experimental import pallas as pl

BN = 256


def _sqrt_ulp_threshold(r):
    bits = jax.lax.bitcast_convert_type(r, jnp.int32)
    rp = jax.lax.bitcast_convert_type(bits + 1, jnp.float32)
    u = rp - r
    h = 0.5 * u
    c = 4097.0 * r
    r_hi = c - (c - r)
    r_lo = r - r_hi
    p_hi = r * r
    p_lo = ((r_hi * r_hi - p_hi) + 2.0 * (r_hi * r_lo)) + r_lo * r_lo
    rh = r * u
    h2 = h * h
    s1 = p_lo + rh
    z = p_hi + (s1 + h2)
    resid = ((p_hi - z) + s1) + h2
    zbits = jax.lax.bitcast_convert_type(z, jnp.int32)
    z_prev = jax.lax.bitcast_convert_type(zbits - 1, jnp.float32)
    t = jnp.where(resid > 0, z, z_prev)
    return jnp.where(r == 0, jnp.float32(0.0), t)


def _device_sqrt_threshold(r):
    bn = r.shape[0]
    tc = _sqrt_ulp_threshold(r)
    tbits = jax.lax.bitcast_convert_type(tc, jnp.int32)
    off = jax.lax.broadcasted_iota(jnp.int32, (bn, 128), 1) - 63
    c = jax.lax.bitcast_convert_type(tbits + off, jnp.float32)
    ok = jnp.sqrt(jnp.maximum(c, 0.0)) <= r
    t = jnp.max(jnp.where(ok, c, -jnp.inf), axis=1, keepdims=True)
    return jnp.where(r == 0, jnp.float32(0.0), t)


def _vq_kernel(t_ref, w_ref, tsq_ref, csq_ref, out_ref):
    k = w_ref.shape[0]
    t = t_ref[...]
    g = jax.lax.dot_general(
        t, w_ref[...], (((1,), (1,)), ((), ())),
        preferred_element_type=jnp.float32)
    d2 = (tsq_ref[...] + csq_ref[...]) + g
    m = jnp.min(d2, axis=1, keepdims=True)
    r = jnp.sqrt(jnp.maximum(m, 0.0))
    thresh = _device_sqrt_threshold(r)
    iota = jax.lax.broadcasted_iota(jnp.int32, d2.shape, 1)
    label = jnp.min(jnp.where(d2 <= thresh, iota, k),
                    axis=1, keepdims=True)
    out_ref[...] = jnp.where(iota == label, 1.0, 0.0).astype(jnp.float32)


def kernel(target, codebook, counts):
    n, d = target.shape
    k = codebook.shape[0]
    cb = codebook / counts[:, None]
    t_sq = jnp.sum(target * target, axis=1, keepdims=True)
    c_sq = jnp.sum(cb * cb, axis=1)[None, :]
    w = -2.0 * cb
    return pl.pallas_call(
        _vq_kernel,
        grid=(n // BN,),
        in_specs=[
            pl.BlockSpec((BN, d), lambda i: (i, 0)),
            pl.BlockSpec((k, d), lambda i: (0, 0)),
            pl.BlockSpec((BN, 1), lambda i: (i, 0)),
            pl.BlockSpec((1, k), lambda i: (0, 0)),
        ],
        out_specs=pl.BlockSpec((BN, k), lambda i: (i, 0)),
        out_shape=jax.ShapeDtypeStruct((n, k), jnp.float32),
    )(target, w, t_sq, c_sq)


# speedup vs baseline: 1.5395x; 1.0051x over previous
import jax
import jax.numpy as jnp
from jax.experimental import pallas as pl

BN = 384


def _sqrt_ulp_threshold(r):
    bits = jax.lax.bitcast_convert_type(r, jnp.int32)
    rp = jax.lax.bitcast_convert_type(bits + 1, jnp.float32)
    u = rp - r
    h = 0.5 * u
    c = 4097.0 * r
    r_hi = c - (c - r)
    r_lo = r - r_hi
    p_hi = r * r
    p_lo = ((r_hi * r_hi - p_hi) + 2.0 * (r_hi * r_lo)) + r_lo * r_lo
    rh = r * u
    h2 = h * h
    s1 = p_lo + rh
    z = p_hi + (s1 + h2)
    resid = ((p_hi - z) + s1) + h2
    zbits = jax.lax.bitcast_convert_type(z, jnp.int32)
    z_prev = jax.lax.bitcast_convert_type(zbits - 1, jnp.float32)
    t = jnp.where(resid > 0, z, z_prev)
    return jnp.where(r == 0, jnp.float32(0.0), t)


def _device_sqrt_threshold(r):
    bn = r.shape[0]
    tc = _sqrt_ulp_threshold(r)
    tbits = jax.lax.bitcast_convert_type(tc, jnp.int32)
    off = jax.lax.broadcasted_iota(jnp.int32, (bn, 128), 1) - 63
    c = jax.lax.bitcast_convert_type(tbits + off, jnp.float32)
    ok = jnp.sqrt(jnp.maximum(c, 0.0)) <= r
    t = jnp.max(jnp.where(ok, c, -jnp.inf), axis=1, keepdims=True)
    return jnp.where(r == 0, jnp.float32(0.0), t)


def _vq_kernel(t_ref, w_ref, tsq_ref, csq_ref, out_ref):
    k = w_ref.shape[0]
    t = t_ref[...]
    g = jax.lax.dot_general(
        t, w_ref[...], (((1,), (1,)), ((), ())),
        preferred_element_type=jnp.float32)
    d2 = (tsq_ref[...] + csq_ref[...]) + g
    m = jnp.min(d2, axis=1, keepdims=True)
    r = jnp.sqrt(jnp.maximum(m, 0.0))
    thresh = _device_sqrt_threshold(r)
    iota = jax.lax.broadcasted_iota(jnp.int32, d2.shape, 1)
    label = jnp.min(jnp.where(d2 <= thresh, iota, k),
                    axis=1, keepdims=True)
    out_ref[...] = jnp.where(iota == label, 1.0, 0.0).astype(jnp.float32)


def kernel(target, codebook, counts):
    n, d = target.shape
    k = codebook.shape[0]
    cb = codebook / counts[:, None]
    t_sq = jnp.sum(target * target, axis=1, keepdims=True)
    c_sq = jnp.sum(cb * cb, axis=1)[None, :]
    w = -2.0 * cb
    return pl.pallas_call(
        _vq_kernel,
        grid=(n // BN,),
        in_specs=[
            pl.BlockSpec((BN, d), lambda i: (i, 0)),
            pl.BlockSpec((k, d), lambda i: (0, 0)),
            pl.BlockSpec((BN, 1), lambda i: (i, 0)),
            pl.BlockSpec((1, k), lambda i: (0, 0)),
        ],
        out_specs=pl.BlockSpec((BN, k), lambda i: (i, 0)),
        out_shape=jax.ShapeDtypeStruct((n, k), jnp.float32),
    )(target, w, t_sq, c_sq)
